# baseline (speedup 1.0000x reference)
import functools

import jax
import jax.numpy as jnp
from jax import lax
from jax.experimental import pallas as pl
from jax.experimental.pallas import tpu as pltpu
from jax.experimental.pallas import tpu_sc as plsc

NUM_BINS = 15
NUM_CLASSES = 19
LANES = 16

B, H, W = 4, 512, 512
NW = 32
WORKERS_PER_BATCH = NW // B
ROWS_PER_WORKER = H // WORKERS_PER_BATCH
TILE_R, TILE_C = 8, 128
COL_TILES = W // TILE_C
NCHUNKS = (ROWS_PER_WORKER // TILE_R) * COL_TILES
GROUPS = TILE_R * TILE_C // LANES
GPR = TILE_C // LANES


def _compute_tile(in_v, out_v, tab_v):
    @plsc.parallel_loop(0, GROUPS, unroll=2)
    def _(g):
        r = lax.shift_right_logical(g, 3)
        off = lax.shift_left(g & (GPR - 1), 4)
        sl = pl.ds(off, LANES)
        v = [in_v[c, r, sl] for c in range(NUM_CLASSES)]
        m = v[0]
        for c in range(1, NUM_CLASSES):
            m = jnp.maximum(m, v[c])
        e = [jnp.exp(v[c] - m) for c in range(NUM_CLASSES)]
        s = e[0]
        for c in range(1, NUM_CLASSES):
            s = s + e[c]
        r15 = jnp.float32(NUM_BINS) / s
        g_vals = []
        for c in range(NUM_CLASSES):
            bin_c = (e[c] * r15).astype(jnp.int32)
            g_vals.append(plsc.load_gather(tab_v, [bin_c + c * LANES]))
        t = g_vals[0]
        for c in range(1, NUM_CLASSES):
            t = t + g_vals[c]
        rn = jnp.float32(1.0) / t
        for c in range(NUM_CLASSES):
            out_v[c, r, sl] = g_vals[c] * rn


def _sc_body(x_hbm, tab_hbm, out_hbm, in0, in1, out0, out1, tab_v,
             si0, si1, so0, so1):
    info = plsc.get_sparse_core_info()
    nc = info.num_cores
    wid = lax.axis_index("s") * nc + lax.axis_index("c")
    batch = wid // WORKERS_PER_BATCH
    row_base = (wid % WORKERS_PER_BATCH) * ROWS_PER_WORKER

    pltpu.sync_copy(tab_hbm, tab_v)

    ins = (in0, in1)
    outs = (out0, out1)
    sis = (si0, si1)
    sos = (so0, so1)

    def _tile_ref(hbm, ch):
        r0 = row_base + lax.shift_right_logical(ch, 2) * TILE_R
        c0 = (ch & (COL_TILES - 1)) * TILE_C
        return hbm.at[batch, :, pl.ds(r0, TILE_R), pl.ds(c0, TILE_C)]

    def src(ch):
        return _tile_ref(x_hbm, ch)

    def dst(ch):
        return _tile_ref(out_hbm, ch)

    pltpu.async_copy(src(0), in0, si0)
    pltpu.async_copy(src(1), in1, si1)

    def ring_body(k, _):
        for b in range(2):
            ch = 2 * k + b
            pltpu.make_async_copy(src(ch), ins[b], sis[b]).wait()

            @pl.when(k >= 1)
            def _():
                pltpu.make_async_copy(outs[b], dst(ch - 2), sos[b]).wait()

            _compute_tile(ins[b], outs[b], tab_v)
            pltpu.async_copy(outs[b], dst(ch), sos[b])

            @pl.when(k < NCHUNKS // 2 - 1)
            def _():
                pltpu.async_copy(src(ch + 2), ins[b], sis[b])
        return 0

    lax.fori_loop(0, NCHUNKS // 2, ring_body, 0)
    last = NCHUNKS - 2
    pltpu.make_async_copy(out0, dst(last), so0).wait()
    pltpu.make_async_copy(out1, dst(last + 1), so1).wait()


@jax.jit
def kernel(logits, val_freqs):
    tab = jnp.concatenate(
        [val_freqs, val_freqs[:, NUM_BINS - 1:]], axis=1).reshape(-1)
    run = pl.kernel(
        _sc_body,
        out_type=jax.ShapeDtypeStruct((B, NUM_CLASSES, H, W), jnp.float32),
        mesh=plsc.VectorSubcoreMesh(core_axis_name="c", subcore_axis_name="s"),
        compiler_params=pltpu.CompilerParams(needs_layout_passes=False),
        scratch_types=[
            pltpu.VMEM((NUM_CLASSES, TILE_R, TILE_C), jnp.float32),
            pltpu.VMEM((NUM_CLASSES, TILE_R, TILE_C), jnp.float32),
            pltpu.VMEM((NUM_CLASSES, TILE_R, TILE_C), jnp.float32),
            pltpu.VMEM((NUM_CLASSES, TILE_R, TILE_C), jnp.float32),
            pltpu.VMEM((NUM_CLASSES * LANES,), jnp.float32),
            pltpu.SemaphoreType.DMA,
            pltpu.SemaphoreType.DMA,
            pltpu.SemaphoreType.DMA,
            pltpu.SemaphoreType.DMA,
        ],
    )
    return run(logits, tab)


# speedup vs baseline: 1.2515x; 1.2515x over previous
import functools

import jax
import jax.numpy as jnp
from jax import lax
from jax.experimental import pallas as pl
from jax.experimental.pallas import tpu as pltpu
from jax.experimental.pallas import tpu_sc as plsc

NUM_BINS = 15
NUM_CLASSES = 19
LANES = 16

B, H, W = 4, 512, 512
NW = 32
WORKERS_PER_BATCH = NW // B
ROWS_PER_WORKER = H // WORKERS_PER_BATCH
TILE_R, TILE_C = 8, 256
NCHUNKS = (ROWS_PER_WORKER // TILE_R) * (W // TILE_C)
GROUPS = TILE_R * TILE_C // LANES
NBUF = 3


def _compute_tile(buf, tab_v):
    @plsc.parallel_loop(0, GROUPS, unroll=1)
    def _(g):
        r = lax.shift_right_logical(g, 4)
        off = lax.shift_left(g & (TILE_C // LANES - 1), 4)
        sl = pl.ds(off, LANES)
        v = [buf[c, r, sl] for c in range(NUM_CLASSES)]
        m = v[0]
        for c in range(1, NUM_CLASSES):
            m = jnp.maximum(m, v[c])
        e = [jnp.exp(v[c] - m) for c in range(NUM_CLASSES)]
        s = e[0]
        for c in range(1, NUM_CLASSES):
            s = s + e[c]
        r15 = jnp.float32(NUM_BINS) / s
        g_vals = []
        for c in range(NUM_CLASSES):
            bin_c = (e[c] * r15).astype(jnp.int32)
            g_vals.append(plsc.load_gather(tab_v, [bin_c + c * LANES]))
        t = g_vals[0]
        for c in range(1, NUM_CLASSES):
            t = t + g_vals[c]
        rn = jnp.float32(1.0) / t
        for c in range(NUM_CLASSES):
            buf[c, r, sl] = g_vals[c] * rn


def _sc_body(x_hbm, tab_hbm, out_hbm, b0, b1, b2, tab_v, si0, si1, si2,
             so0, so1, so2):
    info = plsc.get_sparse_core_info()
    nc = info.num_cores
    wid = lax.axis_index("s") * nc + lax.axis_index("c")
    batch = wid // WORKERS_PER_BATCH
    row_base = (wid % WORKERS_PER_BATCH) * ROWS_PER_WORKER

    pltpu.sync_copy(tab_hbm, tab_v)

    bufs = (b0, b1, b2)
    sis = (si0, si1, si2)
    sos = (so0, so1, so2)

    def _tile_ref(hbm, ch):
        r0 = row_base + lax.shift_right_logical(ch, 1) * TILE_R
        c0 = (ch & 1) * TILE_C
        return hbm.at[batch, :, pl.ds(r0, TILE_R), pl.ds(c0, TILE_C)]

    def src(ch):
        return _tile_ref(x_hbm, ch)

    def dst(ch):
        return _tile_ref(out_hbm, ch)

    pltpu.async_copy(src(0), b0, si0)
    pltpu.async_copy(src(1), b1, si1)

    def ring_body(k, _):
        for j in range(NBUF):
            ch = NBUF * k + j
            pltpu.make_async_copy(src(ch), bufs[j], sis[j]).wait()
            _compute_tile(bufs[j], tab_v)
            pltpu.async_copy(bufs[j], dst(ch), sos[j])
            nb = (j + 2) % NBUF

            def _drain_and_prefetch():
                pltpu.make_async_copy(bufs[nb], dst(ch - 1), sos[nb]).wait()
                pltpu.async_copy(src(ch + 2), bufs[nb], sis[nb])

            if j == 0:
                @pl.when(k >= 1)
                def _():
                    _drain_and_prefetch()

                @pl.when(k < 1)
                def _():
                    pltpu.async_copy(src(ch + 2), bufs[nb], sis[nb])
            elif j == NBUF - 1:
                @pl.when(k < (NCHUNKS - 1) // NBUF - 1)
                def _():
                    _drain_and_prefetch()
            else:
                _drain_and_prefetch()
        return 0

    lax.fori_loop(0, (NCHUNKS - 1) // NBUF, ring_body, 0)

    last = NCHUNKS - 1
    pltpu.make_async_copy(src(last), b0, si0).wait()
    _compute_tile(b0, tab_v)
    pltpu.async_copy(b0, dst(last), so0)
    pltpu.make_async_copy(b1, dst(last - 2), so1).wait()
    pltpu.make_async_copy(b2, dst(last - 1), so2).wait()
    pltpu.make_async_copy(b0, dst(last), so0).wait()


@jax.jit
def kernel(logits, val_freqs):
    tab = jnp.concatenate(
        [val_freqs, val_freqs[:, NUM_BINS - 1:]], axis=1).reshape(-1)
    run = pl.kernel(
        _sc_body,
        out_type=jax.ShapeDtypeStruct((B, NUM_CLASSES, H, W), jnp.float32),
        mesh=plsc.VectorSubcoreMesh(core_axis_name="c", subcore_axis_name="s"),
        compiler_params=pltpu.CompilerParams(needs_layout_passes=False),
        scratch_types=[
            pltpu.VMEM((NUM_CLASSES, TILE_R, TILE_C), jnp.float32),
            pltpu.VMEM((NUM_CLASSES, TILE_R, TILE_C), jnp.float32),
            pltpu.VMEM((NUM_CLASSES, TILE_R, TILE_C), jnp.float32),
            pltpu.VMEM((NUM_CLASSES * LANES,), jnp.float32),
            pltpu.SemaphoreType.DMA,
            pltpu.SemaphoreType.DMA,
            pltpu.SemaphoreType.DMA,
            pltpu.SemaphoreType.DMA,
            pltpu.SemaphoreType.DMA,
            pltpu.SemaphoreType.DMA,
        ],
    )
    return run(logits, tab)
